# Initial kernel scaffold; baseline (speedup 1.0000x reference)
import jax
import jax.numpy as jnp
from jax.experimental import pallas as pl


def kernel(h_paper, h_author, edge_index_writes, edge_index_cites, Wk, bk, Wq, bq, Wv, bv, Wa, ba, rel_att, rel_msg, rel_pri, skip):
    raise NotImplementedError("write your pallas kernel here")


# speedup vs baseline: 3.7471x; 3.7471x over previous
import functools

import jax
import jax.numpy as jnp
from jax import lax
from jax.experimental import pallas as pl
from jax.experimental.pallas import tpu as pltpu
from jax.experimental.pallas import tpu_sc as plsc

IN_DIM = 512
OUT_DIM = 512
H = 8
DK = 64
SQRT_DK = 8.0
N = 10000
NPAD = 10240
E = 150000
NC, NS, LANES = 2, 16, 16
NW = NC * NS
CHUNK = 64
EPW = 4736
EPAD = EPW * NW
NCHUNKS = EPW // CHUNK
ROWS_PER_TILE = NPAD // NS
AW = 144
NPASS = 8


def _mm_body(x_ref, w_ref, b_ref, o_ref):
    o_ref[...] = (
        jnp.dot(x_ref[...], w_ref[...], preferred_element_type=jnp.float32)
        + b_ref[...]
    )


def _mm(x, w, b, mb=1024):
    m, k = x.shape
    n = w.shape[1]
    return pl.pallas_call(
        _mm_body,
        grid=(m // mb,),
        in_specs=[
            pl.BlockSpec((mb, k), lambda i: (i, 0)),
            pl.BlockSpec((k, n), lambda i: (0, 0)),
            pl.BlockSpec((1, n), lambda i: (0, 0)),
        ],
        out_specs=pl.BlockSpec((mb, n), lambda i: (i, 0)),
        out_shape=jax.ShapeDtypeStruct((m, n), jnp.float32),
    )(x, w, b.reshape(1, n))


def _sc_edge_kernel(qt, kt, vt, srcf, dstf, zr, out,
                    sidx, didx, sadj, dadj, kbuf, qbuf, vbuf, obuf,
                    acc, sem0, sem1, sem2):
    c = lax.axis_index("c")
    s = lax.axis_index("s")
    wid = s * NC + c

    def pass_body(q, _):
        pltpu.sync_copy(zr.at[pl.ds(s * ROWS_PER_TILE, ROWS_PER_TILE)],
                        acc.at[pl.ds(s * ROWS_PER_TILE, ROWS_PER_TILE)])
        plsc.subcore_barrier()

        r = q // 4
        off = q * NPAD

        def chunk_body(j, _):
            base = r * EPAD + wid * EPW + j * CHUNK
            pltpu.sync_copy(srcf.at[pl.ds(base, CHUNK)], sidx)
            pltpu.sync_copy(dstf.at[pl.ds(base, CHUNK)], didx)
            for i in range(CHUNK // LANES):
                sl = pl.ds(i * LANES, LANES)
                sadj[sl] = sidx[sl] + off
                dadj[sl] = didx[sl] + off
            h0 = pltpu.async_copy(kt.at[sadj], kbuf, sem0)
            h1 = pltpu.async_copy(qt.at[dadj], qbuf, sem1)
            h2 = pltpu.async_copy(vt.at[sadj], vbuf, sem2)
            h0.wait()
            h1.wait()
            h2.wait()

            def group_body(g, _):
                rows = g * LANES + jnp.arange(LANES, dtype=jnp.int32)

                def col(cc):
                    return jnp.full((LANES,), cc, jnp.int32)

                zeros = jnp.zeros((LANES,), jnp.float32)
                for i in range(130, AW):
                    plsc.store_scatter(obuf, [rows, col(i)], zeros)
                for h in range(2):
                    acc_v = zeros
                    for f in range(64):
                        cv = col(h * 64 + f)
                        acc_v = acc_v + (plsc.load_gather(qbuf, [rows, cv])
                                         * plsc.load_gather(kbuf, [rows, cv]))
                    ex = jnp.exp(acc_v)
                    plsc.store_scatter(obuf, [rows, col(128 + h)], ex)
                    for f in range(64):
                        cv = col(h * 64 + f)
                        msg = plsc.load_gather(vbuf, [rows, cv]) * ex
                        plsc.store_scatter(obuf, [rows, cv], msg)
                return 0

            lax.fori_loop(0, CHUNK // LANES, group_body, 0)
            pltpu.sync_copy(obuf, acc.at[didx], add=True)
            return 0

        lax.fori_loop(0, NCHUNKS, chunk_body, 0)
        plsc.subcore_barrier()
        orow = (q * NC + c) * NPAD + s * ROWS_PER_TILE
        pltpu.sync_copy(acc.at[pl.ds(s * ROWS_PER_TILE, ROWS_PER_TILE)],
                        out.at[pl.ds(orow, ROWS_PER_TILE)])
        plsc.subcore_barrier()
        return 0

    lax.fori_loop(0, NPASS, pass_body, 0)


_sc_edge = pl.kernel(
    _sc_edge_kernel,
    out_type=jax.ShapeDtypeStruct((NPASS * NC * NPAD, AW), jnp.float32),
    mesh=plsc.VectorSubcoreMesh(core_axis_name="c", subcore_axis_name="s",
                                num_cores=NC, num_subcores=NS),
    compiler_params=pltpu.CompilerParams(use_tc_tiling_on_sc=False,
                                         needs_layout_passes=False),
    scratch_types=[
        pltpu.VMEM((CHUNK,), jnp.int32),
        pltpu.VMEM((CHUNK,), jnp.int32),
        pltpu.VMEM((CHUNK,), jnp.int32),
        pltpu.VMEM((CHUNK,), jnp.int32),
        pltpu.VMEM((CHUNK, 128), jnp.float32),
        pltpu.VMEM((CHUNK, 128), jnp.float32),
        pltpu.VMEM((CHUNK, 128), jnp.float32),
        pltpu.VMEM((CHUNK, AW), jnp.float32),
        pltpu.VMEM_SHARED((NPAD, AW), jnp.float32),
        pltpu.SemaphoreType.DMA,
        pltpu.SemaphoreType.DMA,
        pltpu.SemaphoreType.DMA,
    ],
)


def _finish_body(p_ref, hp_ref, wa_ref, ba_ref, sk_ref, o_ref):
    aggs = []
    for r in range(2):
        num_parts = []
        den_parts = []
        for p in range(4):
            a = p_ref[(r * 4 + p) * 2]
            b = p_ref[(r * 4 + p) * 2 + 1]
            num_parts.append(a[:, :128] + b[:, :128])
            den_parts.append(a[:, 128:130] + b[:, 128:130])
        den = jnp.concatenate(den_parts, axis=1)
        rec = 1.0 / jnp.maximum(den, 1e-9)
        head_cols = []
        for h in range(2 * 4):
            num_h = num_parts[h // 2][:, (h % 2) * 64:(h % 2) * 64 + 64]
            head_cols.append(num_h * rec[:, h:h + 1])
        aggs.append(jnp.concatenate(head_cols, axis=1))
    t = 0.5 * (aggs[0] + aggs[1])
    alpha = jax.nn.sigmoid(sk_ref[0, 0])
    trans = (jnp.dot(t, wa_ref[...], preferred_element_type=jnp.float32)
             + ba_ref[...])
    o_ref[...] = trans * alpha + hp_ref[...] * (1.0 - alpha)


def _finish(parts, h_paper, wa_t, ba, skip0, mb=1000):
    return pl.pallas_call(
        _finish_body,
        grid=(N // mb,),
        in_specs=[
            pl.BlockSpec((NPASS * NC, mb, AW), lambda i: (0, i, 0)),
            pl.BlockSpec((mb, OUT_DIM), lambda i: (i, 0)),
            pl.BlockSpec((OUT_DIM, OUT_DIM), lambda i: (0, 0)),
            pl.BlockSpec((1, OUT_DIM), lambda i: (0, 0)),
            pl.BlockSpec(memory_space=pltpu.SMEM),
        ],
        out_specs=pl.BlockSpec((mb, OUT_DIM), lambda i: (i, 0)),
        out_shape=jax.ShapeDtypeStruct((N, OUT_DIM), jnp.float32),
    )(parts, h_paper, wa_t, ba.reshape(1, OUT_DIM), skip0)


def _block_diag(a):
    out = jnp.zeros((H * DK, H * DK), dtype=a.dtype)
    for h in range(H):
        out = out.at[h * DK:(h + 1) * DK, h * DK:(h + 1) * DK].set(a[h])
    return out


def kernel(h_paper, h_author, edge_index_writes, edge_index_cites,
           Wk, bk, Wq, bq, Wv, bv, Wa, ba, rel_att, rel_msg, rel_pri, skip):
    f32 = jnp.float32
    hp = h_paper.astype(f32)
    ha = h_author.astype(f32)

    bd_a0 = _block_diag(rel_att[0])
    bd_a1 = _block_diag(rel_att[1])
    bd_m0 = _block_diag(rel_msg[0])
    bd_m1 = _block_diag(rel_msg[1])
    scale0 = jnp.repeat(rel_pri[0] / SQRT_DK, DK)
    scale1 = jnp.repeat(rel_pri[1] / SQRT_DK, DK)
    wq_t = Wq[0].T
    wp = jnp.concatenate([
        wq_t * scale0[None, :],
        wq_t * scale1[None, :],
        Wk[0].T @ bd_a1,
        Wv[0].T @ bd_m1,
    ], axis=1)
    bp = jnp.concatenate([
        bq[0] * scale0, bq[0] * scale1, bk[0] @ bd_a1, bv[0] @ bd_m1])
    wauth = jnp.concatenate([Wk[1].T @ bd_a0, Wv[1].T @ bd_m0], axis=1)
    bauth = jnp.concatenate([bk[1] @ bd_a0, bv[1] @ bd_m0])

    hp_pad = jnp.pad(hp, ((0, NPAD - N), (0, 0)))
    ha_pad = jnp.pad(ha, ((0, NPAD - N), (0, 0)))

    proj_p = _mm(hp_pad, wp, bp)
    proj_a = _mm(ha_pad, wauth, bauth)

    qt = []
    ktb = []
    vtb = []
    for q in range(NPASS):
        r, p = divmod(q, 4)
        qt.append(proj_p[:, r * 512 + p * 128: r * 512 + (p + 1) * 128])
        if r == 0:
            ktb.append(proj_a[:, p * 128:(p + 1) * 128])
            vtb.append(proj_a[:, 512 + p * 128: 512 + (p + 1) * 128])
        else:
            ktb.append(proj_p[:, 1024 + p * 128: 1024 + (p + 1) * 128])
            vtb.append(proj_p[:, 1536 + p * 128: 1536 + (p + 1) * 128])
    qtf = jnp.concatenate(qt, axis=0)
    ktf = jnp.concatenate(ktb, axis=0)
    vtf = jnp.concatenate(vtb, axis=0)

    def pad_edges(eidx):
        src = eidx[0].astype(jnp.int32)
        dst = eidx[1].astype(jnp.int32)
        src = jnp.pad(src, (0, EPAD - E))
        dst = jnp.pad(dst, (0, EPAD - E), constant_values=N)
        return src, dst

    s0, d0 = pad_edges(edge_index_writes)
    s1, d1 = pad_edges(edge_index_cites)
    srcf = jnp.concatenate([s0, s1])
    dstf = jnp.concatenate([d0, d1])
    zeros_rows = jnp.zeros((NPAD, AW), f32)

    parts = _sc_edge(qtf, ktf, vtf, srcf, dstf, zeros_rows)
    parts = parts.reshape(NPASS * NC, NPAD, AW)

    new_paper = _finish(parts, hp, Wa[0].T, ba[0],
                        skip.astype(f32)[0].reshape(1, 1))
    return new_paper, h_author


# speedup vs baseline: 4.4260x; 1.1812x over previous
import jax
import jax.numpy as jnp
from jax import lax
from jax.experimental import pallas as pl
from jax.experimental.pallas import tpu as pltpu
from jax.experimental.pallas import tpu_sc as plsc

IN_DIM = 512
OUT_DIM = 512
H = 8
DK = 64
SQRT_DK = 8.0
N = 10000
NPAD = 10240
E = 150000
NC, NS, LANES = 2, 16, 16
NW = NC * NS
CHUNK = 128
EPW = 4736
EPAD = EPW * NW
NCHUNKS = EPW // CHUNK
RPT = NPAD // NS
AW = 80
NPASS = 16


def _mm_body(x_ref, w_ref, b_ref, o_ref):
    o_ref[...] = (
        jnp.dot(x_ref[...], w_ref[...], preferred_element_type=jnp.float32)
        + b_ref[...]
    )


def _mm(x, w, b, mb=1024):
    m, k = x.shape
    n = w.shape[1]
    return pl.pallas_call(
        _mm_body,
        grid=(m // mb,),
        in_specs=[
            pl.BlockSpec((mb, k), lambda i: (i, 0)),
            pl.BlockSpec((k, n), lambda i: (0, 0)),
            pl.BlockSpec((1, n), lambda i: (0, 0)),
        ],
        out_specs=pl.BlockSpec((mb, n), lambda i: (i, 0)),
        out_shape=jax.ShapeDtypeStruct((m, n), jnp.float32),
    )(x, w, b.reshape(1, n))


def _sc_edge_kernel(qtf, kv0, kv1, srcf, dstf, zr, out,
                    sidx_all, didx_all, didx_s,
                    kvadj0, kvadj1, qadj0, qadj1,
                    kvb0, kvb1, qb0, qb1, obuf,
                    acc, semkv0, semkv1, semq0, semq1):
    c = lax.axis_index("c")
    s = lax.axis_index("s")
    wid = s * NC + c

    kvb = (kvb0, kvb1)
    qb = (qb0, qb1)
    kvadj = (kvadj0, kvadj1)
    qadj = (qadj0, qadj1)
    semkv = (semkv0, semkv1)
    semq = (semq0, semq1)

    def col(cc):
        return jnp.full((LANES,), cc, jnp.int32)

    def zinit(g, _):
        rows = g * LANES + jnp.arange(LANES, dtype=jnp.int32)
        zv = jnp.zeros((LANES,), jnp.float32)
        for i in range(AW - 15, AW):
            plsc.store_scatter(obuf, [rows, col(i)], zv)
        return 0

    lax.fori_loop(0, CHUNK // LANES, zinit, 0)

    for r in range(2):
        kvt = kv0 if r == 0 else kv1
        pltpu.sync_copy(srcf.at[pl.ds(r * EPAD + wid * EPW, EPW)], sidx_all)
        pltpu.sync_copy(dstf.at[pl.ds(r * EPAD + wid * EPW, EPW)], didx_all)

        def head_body(h, _, kvt=kvt, r=r):
            pltpu.sync_copy(zr.at[pl.ds(s * RPT, RPT)],
                            acc.at[pl.ds(s * RPT, RPT)])
            plsc.subcore_barrier()

            qrow = r * 8 + h

            def build_idx(b, j):
                for i in range(CHUNK // LANES):
                    sl = pl.ds(i * LANES, LANES)
                    src_v = sidx_all[pl.ds(j * CHUNK + i * LANES, LANES)]
                    dst_v = didx_all[pl.ds(j * CHUNK + i * LANES, LANES)]
                    kvadj[b][sl] = src_v * 8 + h
                    qadj[b][sl] = dst_v * 16 + qrow

            def issue(b):
                pltpu.async_copy(kvt.at[kvadj[b]], kvb[b], semkv[b])
                pltpu.async_copy(qtf.at[qadj[b]], qb[b], semq[b])

            def wait(b):
                pltpu.make_async_copy(kvt.at[kvadj[b]], kvb[b],
                                      semkv[b]).wait()
                pltpu.make_async_copy(qtf.at[qadj[b]], qb[b], semq[b]).wait()

            def process(b, j):
                wait(b)

                def group(g, _):
                    rows = g * LANES + jnp.arange(LANES, dtype=jnp.int32)
                    acc_v = jnp.zeros((LANES,), jnp.float32)
                    for f in range(DK):
                        cv = col(f)
                        acc_v = acc_v + (plsc.load_gather(qb[b], [rows, cv])
                                         * plsc.load_gather(kvb[b],
                                                            [rows, cv]))
                    ex = jnp.exp(acc_v)
                    plsc.store_scatter(obuf, [rows, col(DK)], ex)
                    for f in range(DK):
                        msg = plsc.load_gather(kvb[b],
                                               [rows, col(DK + f)]) * ex
                        plsc.store_scatter(obuf, [rows, col(f)], msg)
                    return 0

                lax.fori_loop(0, CHUNK // LANES, group, 0)
                for i in range(CHUNK // LANES):
                    sl = pl.ds(i * LANES, LANES)
                    didx_s[sl] = didx_all[pl.ds(j * CHUNK + i * LANES, LANES)]
                pltpu.sync_copy(obuf, acc.at[didx_s], add=True)

            build_idx(0, 0)
            issue(0)

            def pair_body(t, _):
                j1 = 2 * t + 1
                j2 = 2 * t + 2

                @pl.when(j1 < NCHUNKS)
                def _():
                    build_idx(1, j1)
                    issue(1)

                process(0, 2 * t)

                @pl.when(j2 < NCHUNKS)
                def _():
                    build_idx(0, j2)
                    issue(0)

                @pl.when(j1 < NCHUNKS)
                def _():
                    process(1, j1)

                return 0

            lax.fori_loop(0, (NCHUNKS + 1) // 2, pair_body, 0)
            plsc.subcore_barrier()
            orow = ((r * 8 + h) * NC + c) * NPAD + s * RPT
            pltpu.sync_copy(acc.at[pl.ds(s * RPT, RPT)],
                            out.at[pl.ds(orow, RPT)])
            plsc.subcore_barrier()
            return 0

        lax.fori_loop(0, 8, head_body, 0)


_sc_edge = pl.kernel(
    _sc_edge_kernel,
    out_type=jax.ShapeDtypeStruct((NPASS * NC * NPAD, AW), jnp.float32),
    mesh=plsc.VectorSubcoreMesh(core_axis_name="c", subcore_axis_name="s",
                                num_cores=NC, num_subcores=NS),
    compiler_params=pltpu.CompilerParams(use_tc_tiling_on_sc=False,
                                         needs_layout_passes=False),
    scratch_types=[
        pltpu.VMEM((EPW,), jnp.int32),
        pltpu.VMEM((EPW,), jnp.int32),
        pltpu.VMEM((CHUNK,), jnp.int32),
        pltpu.VMEM((CHUNK,), jnp.int32),
        pltpu.VMEM((CHUNK,), jnp.int32),
        pltpu.VMEM((CHUNK,), jnp.int32),
        pltpu.VMEM((CHUNK,), jnp.int32),
        pltpu.VMEM((CHUNK, 2 * DK), jnp.float32),
        pltpu.VMEM((CHUNK, 2 * DK), jnp.float32),
        pltpu.VMEM((CHUNK, DK), jnp.float32),
        pltpu.VMEM((CHUNK, DK), jnp.float32),
        pltpu.VMEM((CHUNK, AW), jnp.float32),
        pltpu.VMEM_SHARED((NPAD, AW), jnp.float32),
        pltpu.SemaphoreType.DMA,
        pltpu.SemaphoreType.DMA,
        pltpu.SemaphoreType.DMA,
        pltpu.SemaphoreType.DMA,
    ],
)


def _finish_body(p_ref, hp_ref, wa_ref, ba_ref, sk_ref, o_ref):
    aggs = []
    for r in range(2):
        head_cols = []
        for h in range(H):
            a = p_ref[(r * 8 + h) * 2]
            b = p_ref[(r * 8 + h) * 2 + 1]
            num = a[:, :DK] + b[:, :DK]
            den = a[:, DK:DK + 1] + b[:, DK:DK + 1]
            head_cols.append(num / jnp.maximum(den, 1e-9))
        aggs.append(jnp.concatenate(head_cols, axis=1))
    t = 0.5 * (aggs[0] + aggs[1])
    alpha = jax.nn.sigmoid(sk_ref[0, 0])
    trans = (jnp.dot(t, wa_ref[...], preferred_element_type=jnp.float32)
             + ba_ref[...])
    o_ref[...] = trans * alpha + hp_ref[...] * (1.0 - alpha)


def _finish(parts, h_paper, wa_t, ba, skip0, mb=1000):
    return pl.pallas_call(
        _finish_body,
        grid=(N // mb,),
        in_specs=[
            pl.BlockSpec((NPASS * NC, mb, AW), lambda i: (0, i, 0)),
            pl.BlockSpec((mb, OUT_DIM), lambda i: (i, 0)),
            pl.BlockSpec((OUT_DIM, OUT_DIM), lambda i: (0, 0)),
            pl.BlockSpec((1, OUT_DIM), lambda i: (0, 0)),
            pl.BlockSpec(memory_space=pltpu.SMEM),
        ],
        out_specs=pl.BlockSpec((mb, OUT_DIM), lambda i: (i, 0)),
        out_shape=jax.ShapeDtypeStruct((N, OUT_DIM), jnp.float32),
    )(parts, h_paper, wa_t, ba.reshape(1, OUT_DIM), skip0)


def _block_diag(a):
    out = jnp.zeros((H * DK, H * DK), dtype=a.dtype)
    for h in range(H):
        out = out.at[h * DK:(h + 1) * DK, h * DK:(h + 1) * DK].set(a[h])
    return out


def _interleave_kv(wk, wv):
    k3 = wk.reshape(IN_DIM, H, DK)
    v3 = wv.reshape(IN_DIM, H, DK)
    return jnp.concatenate([k3, v3], axis=2).reshape(IN_DIM, 2 * OUT_DIM)


def _interleave_kv_b(bk_, bv_):
    k2 = bk_.reshape(H, DK)
    v2 = bv_.reshape(H, DK)
    return jnp.concatenate([k2, v2], axis=1).reshape(2 * OUT_DIM)


def kernel(h_paper, h_author, edge_index_writes, edge_index_cites,
           Wk, bk, Wq, bq, Wv, bv, Wa, ba, rel_att, rel_msg, rel_pri, skip):
    f32 = jnp.float32
    hp = h_paper.astype(f32)
    ha = h_author.astype(f32)

    bd_a0 = _block_diag(rel_att[0])
    bd_a1 = _block_diag(rel_att[1])
    bd_m0 = _block_diag(rel_msg[0])
    bd_m1 = _block_diag(rel_msg[1])
    scale0 = jnp.repeat(rel_pri[0] / SQRT_DK, DK)
    scale1 = jnp.repeat(rel_pri[1] / SQRT_DK, DK)
    wq_t = Wq[0].T
    wp = jnp.concatenate([
        wq_t * scale0[None, :],
        wq_t * scale1[None, :],
        _interleave_kv(Wk[0].T @ bd_a1, Wv[0].T @ bd_m1),
    ], axis=1)
    bp = jnp.concatenate([
        bq[0] * scale0, bq[0] * scale1,
        _interleave_kv_b(bk[0] @ bd_a1, bv[0] @ bd_m1)])
    wauth = _interleave_kv(Wk[1].T @ bd_a0, Wv[1].T @ bd_m0)
    bauth = _interleave_kv_b(bk[1] @ bd_a0, bv[1] @ bd_m0)

    hp_pad = jnp.pad(hp, ((0, NPAD - N), (0, 0)))
    ha_pad = jnp.pad(ha, ((0, NPAD - N), (0, 0)))

    proj_p = _mm(hp_pad, wp, bp)
    proj_a = _mm(ha_pad, wauth, bauth)

    qtf = proj_p[:, :1024].reshape(NPAD * 16, DK)
    kv1 = proj_p[:, 1024:].reshape(NPAD * 8, 2 * DK)
    kv0 = proj_a.reshape(NPAD * 8, 2 * DK)

    def pad_edges(eidx):
        src = eidx[0].astype(jnp.int32)
        dst = eidx[1].astype(jnp.int32)
        src = jnp.pad(src, (0, EPAD - E))
        dst = jnp.pad(dst, (0, EPAD - E), constant_values=N)
        return src, dst

    s0, d0 = pad_edges(edge_index_writes)
    s1, d1 = pad_edges(edge_index_cites)
    srcf = jnp.concatenate([s0, s1])
    dstf = jnp.concatenate([d0, d1])
    zeros_rows = jnp.zeros((NPAD, AW), f32)

    parts = _sc_edge(qtf, kv0, kv1, srcf, dstf, zeros_rows)
    parts = parts.reshape(NPASS * NC, NPAD, AW)

    new_paper = _finish(parts, hp, Wa[0].T, ba[0],
                        skip.astype(f32)[0].reshape(1, 1))
    return new_paper, h_author
